# Initial kernel scaffold; baseline (speedup 1.0000x reference)
import jax
import jax.numpy as jnp
from jax.experimental import pallas as pl


def kernel(idx, frames):
    raise NotImplementedError("write your pallas kernel here")


# speedup vs baseline: 10.8964x; 10.8964x over previous
import functools

import jax
import jax.numpy as jnp
from jax import lax
from jax.experimental import pallas as pl
from jax.experimental.pallas import tpu as pltpu
from jax.experimental.pallas import tpu_sc as plsc

K = 16
D = K * K
CHUNK = 128


@functools.cache
def _make_gather(V: int, B: int):
  info = plsc.get_sparse_core_info()
  nc, ns = info.num_cores, info.num_subcores
  nw = nc * ns
  assert B % (nw * CHUNK) == 0
  b_per_w = B // nw
  n_chunks = b_per_w // CHUNK
  mesh = plsc.VectorSubcoreMesh(core_axis_name="c", subcore_axis_name="s")

  @functools.partial(
      pl.kernel,
      out_type=jax.ShapeDtypeStruct((B, D), jnp.float32),
      mesh=mesh,
      scratch_types=[
          pltpu.VMEM((n_chunks, CHUNK), jnp.int32),
          pltpu.VMEM((CHUNK, D), jnp.float32),
          pltpu.VMEM((CHUNK, D), jnp.float32),
          pltpu.SemaphoreType.DMA,
          pltpu.SemaphoreType.DMA,
      ],
  )
  def gather(table_hbm, idx_hbm, out_hbm, idx_v, rows0, rows1, sem0, sem1):
    wid = lax.axis_index("s") * nc + lax.axis_index("c")
    base = wid * b_per_w
    chunk0 = wid * n_chunks
    pltpu.sync_copy(idx_hbm.at[pl.ds(chunk0, n_chunks)], idx_v)
    bufs = (rows0, rows1)
    sems = (sem0, sem1)
    copies = [None, None]
    copies[0] = pltpu.async_copy(table_hbm.at[idx_v.at[0]], rows0, sem0)
    for j in range(n_chunks):
      b = j % 2
      copies[b].wait()
      if j + 1 < n_chunks:
        nb = (j + 1) % 2
        copies[nb] = pltpu.async_copy(
            table_hbm.at[idx_v.at[j + 1]], bufs[nb], sems[nb])
      pltpu.sync_copy(bufs[b], out_hbm.at[pl.ds(base + j * CHUNK, CHUNK)])

  return gather


def kernel(idx, frames):
  V = frames.shape[0]
  B = idx.shape[0]
  table = frames.reshape(V, D)
  idx2d = idx.astype(jnp.int32).reshape(B // CHUNK, CHUNK)
  out = _make_gather(V, B)(table, idx2d)
  return out.reshape(B, K, K)


# speedup vs baseline: 10.9697x; 1.0067x over previous
import functools

import jax
import jax.numpy as jnp
from jax import lax
from jax.experimental import pallas as pl
from jax.experimental.pallas import tpu as pltpu
from jax.experimental.pallas import tpu_sc as plsc

K = 16
D = K * K
CHUNK = 128


@functools.cache
def _make_gather(V: int, B: int):
  info = plsc.get_sparse_core_info()
  nc, ns = info.num_cores, info.num_subcores
  nw = nc * ns
  assert B % (nw * CHUNK) == 0
  b_per_w = B // nw
  n_chunks = b_per_w // CHUNK
  mesh = plsc.VectorSubcoreMesh(core_axis_name="c", subcore_axis_name="s")

  @functools.partial(
      pl.kernel,
      out_type=jax.ShapeDtypeStruct((B, D), jnp.float32),
      mesh=mesh,
      scratch_types=[
          pltpu.VMEM((n_chunks, CHUNK), jnp.int32),
          pltpu.VMEM((3, CHUNK, D), jnp.float32),
          pltpu.SemaphoreType.DMA,
          pltpu.SemaphoreType.DMA,
          pltpu.SemaphoreType.DMA,
          pltpu.SemaphoreType.DMA,
          pltpu.SemaphoreType.DMA,
          pltpu.SemaphoreType.DMA,
      ],
  )
  def gather(table_hbm, idx_hbm, out_hbm, idx_v, rows, g0, g1, g2, s0, s1, s2):
    wid = lax.axis_index("s") * nc + lax.axis_index("c")
    base = wid * b_per_w
    chunk0 = wid * n_chunks
    pltpu.sync_copy(idx_hbm.at[pl.ds(chunk0, n_chunks)], idx_v)
    gsems = (g0, g1, g2)
    ssems = (s0, s1, s2)
    gathers = [None] * n_chunks
    stores = [None] * n_chunks
    for j in range(min(2, n_chunks)):
      gathers[j] = pltpu.async_copy(
          table_hbm.at[idx_v.at[j]], rows.at[j % 3], gsems[j % 3])
    for j in range(n_chunks):
      b = j % 3
      gathers[j].wait()
      stores[j] = pltpu.async_copy(
          rows.at[b], out_hbm.at[pl.ds(base + j * CHUNK, CHUNK)], ssems[b])
      jj = j + 2
      if jj < n_chunks:
        if jj - 3 >= 0:
          stores[jj - 3].wait()
        gathers[jj] = pltpu.async_copy(
            table_hbm.at[idx_v.at[jj]], rows.at[jj % 3], gsems[jj % 3])
    for j in range(max(0, n_chunks - 3), n_chunks):
      stores[j].wait()

  return gather


def kernel(idx, frames):
  V = frames.shape[0]
  B = idx.shape[0]
  table = frames.reshape(V, D)
  idx2d = idx.astype(jnp.int32).reshape(B // CHUNK, CHUNK)
  out = _make_gather(V, B)(table, idx2d)
  return out.reshape(B, K, K)


# speedup vs baseline: 18.3724x; 1.6748x over previous
import functools

import jax
import jax.numpy as jnp
from jax import lax
from jax.experimental import pallas as pl
from jax.experimental.pallas import tpu as pltpu
from jax.experimental.pallas import tpu_sc as plsc

K = 16
D = K * K
L = 16
OUT_CHUNK = 4096


@functools.cache
def _make_gather(V: int, B: int):
  info = plsc.get_sparse_core_info()
  nc, ns = info.num_cores, info.num_subcores
  nw = nc * ns
  assert D % nw == 0
  d_per_w = D // nw
  n_flush = B // OUT_CHUNK
  groups_per_flush = OUT_CHUNK // L
  mesh = plsc.VectorSubcoreMesh(core_axis_name="c", subcore_axis_name="s")

  @functools.partial(
      pl.kernel,
      out_type=jax.ShapeDtypeStruct((D, B), jnp.float32),
      mesh=mesh,
      compiler_params=pltpu.CompilerParams(needs_layout_passes=False),
      scratch_types=[
          pltpu.VMEM((V,), jnp.float32),
          pltpu.VMEM((B,), jnp.int32),
          pltpu.VMEM((2, OUT_CHUNK), jnp.float32),
          pltpu.SemaphoreType.DMA,
          pltpu.SemaphoreType.DMA,
      ],
  )
  def gather(table_t, idx_hbm, out_t, col_v, idx_v, obuf, osem0, osem1):
    wid = lax.axis_index("s") * nc + lax.axis_index("c")
    pltpu.sync_copy(idx_hbm, idx_v)
    osems = (osem0, osem1)
    for j in range(d_per_w):
      d = wid * d_per_w + j
      pltpu.sync_copy(table_t.at[d], col_v)
      stores = [None, None]
      for h in range(n_flush):
        hb = h % 2
        def body(g, h=h, hb=hb):
          iv = idx_v[pl.ds(h * OUT_CHUNK + g * L, L)]
          obuf[hb, pl.ds(g * L, L)] = plsc.load_gather(col_v, [iv])
        plsc.parallel_loop(0, groups_per_flush, 1, unroll=8)(body)
        if stores[hb] is not None:
          stores[hb].wait()
        stores[hb] = pltpu.async_copy(
            obuf.at[hb], out_t.at[d, pl.ds(h * OUT_CHUNK, OUT_CHUNK)],
            osems[hb])
      for st in stores:
        if st is not None:
          st.wait()

  return gather


def kernel(idx, frames):
  V = frames.shape[0]
  B = idx.shape[0]
  table_t = frames.reshape(V, D).T
  out_t = _make_gather(V, B)(table_t, idx.astype(jnp.int32))
  return out_t.T.reshape(B, K, K)
